# baseline (speedup 1.0000x reference)
import functools
import math

import jax
import jax.numpy as jnp
from jax import lax
from jax.experimental import pallas as pl
from jax.experimental.pallas import tpu as pltpu
from jax.experimental.pallas import tpu_sc as plsc

_IMG_H = 800.0
_IMG_W = 800.0
_PRE_NMS = 2000
_POST_NMS = 1000
_NMS_THRESH = 0.7
_N = 20000
_NPAD = 20480
_R = 160
_C = 128
_CPAD = 2048
_SPAD = 2304
_DUMP = 2302
_OPAD = 1024
_BBOX_XFORM_CLIP = math.log(1000.0 / 16)
_NTILES = 32
_CHUNK = _NPAD // _NTILES


def _decode_select_kernel(anch_ref, delt_ref, obj_ref, data_ref, idx_ref):
    ax1 = anch_ref[0 * _R:1 * _R, :]
    ay1 = anch_ref[1 * _R:2 * _R, :]
    ax2 = anch_ref[2 * _R:3 * _R, :]
    ay2 = anch_ref[3 * _R:4 * _R, :]
    dx = delt_ref[0 * _R:1 * _R, :]
    dy = delt_ref[1 * _R:2 * _R, :]
    dw = delt_ref[2 * _R:3 * _R, :]
    dh = delt_ref[3 * _R:4 * _R, :]
    obj = obj_ref[...]

    widths = ax2 - ax1
    heights = ay2 - ay1
    ctr_x = ax1 + 0.5 * widths
    ctr_y = ay1 + 0.5 * heights
    dw = jnp.minimum(dw, _BBOX_XFORM_CLIP)
    dh = jnp.minimum(dh, _BBOX_XFORM_CLIP)
    pred_ctr_x = dx * widths + ctr_x
    pred_ctr_y = dy * heights + ctr_y
    pred_w = jnp.exp(dw) * widths
    pred_h = jnp.exp(dh) * heights

    x1 = jnp.clip(pred_ctr_x - 0.5 * pred_w, 0.0, _IMG_W)
    y1 = jnp.clip(pred_ctr_y - 0.5 * pred_h, 0.0, _IMG_H)
    x2 = jnp.clip(pred_ctr_x + 0.5 * pred_w, 0.0, _IMG_W)
    y2 = jnp.clip(pred_ctr_y + 0.5 * pred_h, 0.0, _IMG_H)

    row = lax.broadcasted_iota(jnp.int32, (_R, _C), 0)
    col = lax.broadcasted_iota(jnp.int32, (_R, _C), 1)
    elem = row * _C + col
    valid = ((x2 - x1) * (y2 - y1) > 1.0) & (elem < _N)
    score = jnp.where(valid, obj, -jnp.inf)

    data_ref[0 * _R:1 * _R, :] = x1
    data_ref[1 * _R:2 * _R, :] = y1
    data_ref[2 * _R:3 * _R, :] = x2
    data_ref[3 * _R:4 * _R, :] = y2
    data_ref[4 * _R:5 * _R, :] = score

    sclean = jnp.where(score == 0.0, 0.0, score)
    bits = lax.bitcast_convert_type(sclean, jnp.uint32)
    flip = jnp.where(bits >= jnp.uint32(0x80000000),
                     jnp.uint32(0xFFFFFFFF), jnp.uint32(0x80000000))
    key = bits ^ flip

    def srch(b, t):
        cand = t | (jnp.uint32(1) << jnp.uint32(31 - b))
        cnt = jnp.sum((key >= cand).astype(jnp.float32))
        return jnp.where(cnt >= float(_PRE_NMS), cand, t)

    t_key = lax.fori_loop(0, 32, srch, jnp.uint32(0))

    gt = (key > t_key).astype(jnp.float32)
    eq = (key == t_key).astype(jnp.float32)
    n_gt = jnp.sum(gt)
    fill = float(_PRE_NMS) - n_gt

    ucol = lax.broadcasted_iota(jnp.int32, (_C, _C), 1)
    urow = lax.broadcasted_iota(jnp.int32, (_C, _C), 0)
    ustrict = (urow < ucol).astype(jnp.float32)
    lcol = lax.broadcasted_iota(jnp.int32, (_R, _R), 1)
    lrow = lax.broadcasted_iota(jnp.int32, (_R, _R), 0)
    lstrict = (lcol < lrow).astype(jnp.float32)
    ones_c = jnp.ones((_C, 1), jnp.float32)

    def excl_prefix(v):
        inrow = lax.dot_general(v, ustrict, (((1,), (0,)), ((), ())),
                                preferred_element_type=jnp.float32)
        rowsum = lax.dot_general(v, ones_c, (((1,), (0,)), ((), ())),
                                 preferred_element_type=jnp.float32)
        rowpref = lax.dot_general(lstrict, rowsum, (((1,), (0,)), ((), ())),
                                  preferred_element_type=jnp.float32)
        return inrow + rowpref

    tie_rank = excl_prefix(eq)
    sel = gt + eq * (tie_rank < fill).astype(jnp.float32)
    dest = excl_prefix(sel)
    idx = jnp.where(sel > 0.0, dest, float(_DUMP)).astype(jnp.int32)
    idx_ref[...] = idx


def _nms_kernel(cand_ref, candt_ref, out_ref, a_ref):
    x1r = cand_ref[0:1, :]
    y1r = cand_ref[1:2, :]
    x2r = cand_ref[2:3, :]
    y2r = cand_ref[3:4, :]
    scr = cand_ref[4:5, :]
    x1c = candt_ref[:, 0:1]
    y1c = candt_ref[:, 1:2]
    x2c = candt_ref[:, 2:3]
    y2c = candt_ref[:, 3:4]
    sc_c = candt_ref[:, 4:5]

    area_r = (x2r - x1r) * (y2r - y1r)
    area_c = (x2c - x1c) * (y2c - y1c)

    col = lax.broadcasted_iota(jnp.int32, (1, _CPAD), 1)
    row = lax.broadcasted_iota(jnp.int32, (_CPAD, 1), 0)

    blk = 256
    for b in range(_CPAD // blk):
        r0 = b * blk
        bx1 = x1c[r0:r0 + blk, :]
        by1 = y1c[r0:r0 + blk, :]
        bx2 = x2c[r0:r0 + blk, :]
        by2 = y2c[r0:r0 + blk, :]
        bsc = sc_c[r0:r0 + blk, :]
        barea = area_c[r0:r0 + blk, :]
        ltx = jnp.maximum(bx1, x1r)
        lty = jnp.maximum(by1, y1r)
        rbx = jnp.minimum(bx2, x2r)
        rby = jnp.minimum(by2, y2r)
        wx = jnp.clip(rbx - ltx, 0.0, None)
        wy = jnp.clip(rby - lty, 0.0, None)
        inter = wx * wy
        union = barea + area_r - inter
        iou = inter / jnp.maximum(union, 1e-9)
        brow = row[r0:r0 + blk, :]
        outranks = (scr > bsc) | ((scr == bsc) & (col < brow))
        mask = ((iou > _NMS_THRESH) & outranks
                & (col < _PRE_NMS) & (brow < _PRE_NMS))
        a_ref[r0:r0 + blk, :] = mask.astype(jnp.float32)

    valid_row = (row < _PRE_NMS).astype(jnp.float32)

    def cond(carry):
        _, changed, it = carry
        return changed & (it < _CPAD + 2)

    def body(carry):
        k, _, it = carry
        s = lax.dot_general(a_ref[...], k, (((1,), (0,)), ((), ())),
                            preferred_element_type=jnp.float32)
        k_new = jnp.where(s > 0.0, 0.0, valid_row)
        changed = jnp.any(k_new != k)
        return k_new, changed, it + 1

    keep, _, _ = lax.while_loop(cond, body, (valid_row, jnp.bool_(True), jnp.int32(0)))

    m = (keep > 0.0) & (sc_c > -jnp.inf) & (row < _PRE_NMS)
    nm = (~m) & (row < _PRE_NMS)
    mf = m.astype(jnp.float32)
    nmf = nm.astype(jnp.float32)

    for b in range(_CPAD // blk):
        r0 = b * blk
        brow = row[r0:r0 + blk, :]
        bsc = sc_c[r0:r0 + blk, :]
        pm = (((scr > bsc) | ((scr == bsc) & (col < brow)))
              & (col < _PRE_NMS) & (brow < _PRE_NMS))
        a_ref[r0:r0 + blk, :] = pm.astype(jnp.float32)

    cnt_m = lax.dot_general(a_ref[...], mf, (((1,), (0,)), ((), ())),
                            preferred_element_type=jnp.float32)

    for b in range(_CPAD // blk):
        r0 = b * blk
        brow = row[r0:r0 + blk, :]
        a_ref[r0:r0 + blk, :] = ((col < brow) & (col < _PRE_NMS)).astype(jnp.float32)

    cnt_nm = lax.dot_general(a_ref[...], nmf, (((1,), (0,)), ((), ())),
                             preferred_element_type=jnp.float32)

    n_m = jnp.sum(mf)
    rank = jnp.where(m, cnt_m, n_m + cnt_nm)
    rank = jnp.where(row < _PRE_NMS, rank, 2.0 * _CPAD)

    out_col = lax.broadcasted_iota(jnp.int32, (1, _OPAD), 1).astype(jnp.float32)
    sel = (rank == out_col) & (out_col < _POST_NMS)

    neg = -jnp.inf
    score_val = jnp.where(m, sc_c, neg)
    out_ref[0:1, :] = jnp.max(jnp.where(sel, x1c, neg), axis=0, keepdims=True)
    out_ref[1:2, :] = jnp.max(jnp.where(sel, y1c, neg), axis=0, keepdims=True)
    out_ref[2:3, :] = jnp.max(jnp.where(sel, x2c, neg), axis=0, keepdims=True)
    out_ref[3:4, :] = jnp.max(jnp.where(sel, y2c, neg), axis=0, keepdims=True)
    out_ref[4:5, :] = jnp.max(jnp.where(sel, score_val, neg), axis=0, keepdims=True)
    out_ref[5:8, :] = jnp.zeros_like(out_ref[5:8, :])


@functools.lru_cache(maxsize=1)
def _make_sc_scatter():
    mesh = plsc.VectorSubcoreMesh(core_axis_name="c", subcore_axis_name="s")
    col_ty = jax.ShapeDtypeStruct((_SPAD,), jnp.float32)

    @functools.partial(
        pl.kernel, mesh=mesh,
        out_type=tuple(col_ty for _ in range(5)),
        scratch_types=[
            pltpu.VMEM((_CHUNK,), jnp.int32),
            pltpu.VMEM((_CHUNK,), jnp.float32),
            pltpu.VMEM((_CHUNK,), jnp.float32),
            pltpu.VMEM((_CHUNK,), jnp.float32),
            pltpu.VMEM((_CHUNK,), jnp.float32),
            pltpu.VMEM((_CHUNK,), jnp.float32),
            pltpu.SemaphoreType.DMA,
        ],
    )
    def sc_scatter(idx_hbm, x1_hbm, y1_hbm, x2_hbm, y2_hbm, sc_hbm,
                   o0, o1, o2, o3, o4, idx_v, v0, v1, v2, v3, v4, sem):
        wid = lax.axis_index("s") * 2 + lax.axis_index("c")
        base = wid * _CHUNK
        pltpu.sync_copy(idx_hbm.at[pl.ds(base, _CHUNK)], idx_v)
        pltpu.sync_copy(x1_hbm.at[pl.ds(base, _CHUNK)], v0)
        pltpu.sync_copy(y1_hbm.at[pl.ds(base, _CHUNK)], v1)
        pltpu.sync_copy(x2_hbm.at[pl.ds(base, _CHUNK)], v2)
        pltpu.sync_copy(y2_hbm.at[pl.ds(base, _CHUNK)], v3)
        pltpu.sync_copy(sc_hbm.at[pl.ds(base, _CHUNK)], v4)
        pltpu.async_copy(v0, o0.at[idx_v], sem).wait()
        pltpu.async_copy(v1, o1.at[idx_v], sem).wait()
        pltpu.async_copy(v2, o2.at[idx_v], sem).wait()
        pltpu.async_copy(v3, o3.at[idx_v], sem).wait()
        pltpu.async_copy(v4, o4.at[idx_v], sem).wait()

    return sc_scatter


def _decode_select(anchors, pred_bbox_deltas, objectness, interpret=False):
    f32 = jnp.float32
    n_extra = _NPAD - _N
    at = anchors.T.astype(f32)
    dt = pred_bbox_deltas.T.astype(f32)
    at = jnp.pad(at, ((0, 0), (0, n_extra))).reshape(4 * _R, _C)
    dt = jnp.pad(dt, ((0, 0), (0, n_extra))).reshape(4 * _R, _C)
    ob = jnp.pad(objectness.astype(f32), (0, n_extra)).reshape(_R, _C)

    data, idx = pl.pallas_call(
        _decode_select_kernel,
        out_shape=(jax.ShapeDtypeStruct((5 * _R, _C), f32),
                   jax.ShapeDtypeStruct((_R, _C), jnp.int32)),
        interpret=interpret,
    )(at, dt, ob)
    return data, idx


def _nms(cols, interpret=False):
    f32 = jnp.float32
    cand = jnp.stack([c[:_CPAD] for c in cols])
    cand = jnp.pad(cand, ((0, 3), (0, 0)))
    candt = cand.T
    out = pl.pallas_call(
        _nms_kernel,
        out_shape=jax.ShapeDtypeStruct((8, _OPAD), f32),
        scratch_shapes=[pltpu.VMEM((_CPAD, _CPAD), f32)],
        interpret=interpret,
    )(cand, candt)
    return out[0:5, :_POST_NMS].T


@jax.jit
def _run(anchors, pred_bbox_deltas, objectness):
    data, idx = _decode_select(anchors, pred_bbox_deltas, objectness)
    planes = [data[i * _R:(i + 1) * _R, :].reshape(_NPAD) for i in range(5)]
    cols = _make_sc_scatter()(idx.reshape(_NPAD), *planes)
    return _nms(cols)


def kernel(anchors, pred_bbox_deltas, objectness):
    return _run(anchors, pred_bbox_deltas, objectness)


# speedup vs baseline: 18.9405x; 18.9405x over previous
import functools
import math

import jax
import jax.numpy as jnp
from jax import lax
from jax.experimental import pallas as pl
from jax.experimental.pallas import tpu as pltpu
from jax.experimental.pallas import tpu_sc as plsc

_IMG_H = 800.0
_IMG_W = 800.0
_PRE_NMS = 2000
_POST_NMS = 1000
_NMS_THRESH = 0.7
_N = 20000
_NPAD = 20480
_R = 160
_C = 128
_CPAD = 2048
_SPAD = 2048 + 20480
_OPAD = 1024
_BBOX_XFORM_CLIP = math.log(1000.0 / 16)
_NTILES = 32
_CHUNK = _NPAD // _NTILES


def _decode_select_kernel(anch_ref, delt_ref, obj_ref, data_ref, idx_ref):
    ax1 = anch_ref[0 * _R:1 * _R, :]
    ay1 = anch_ref[1 * _R:2 * _R, :]
    ax2 = anch_ref[2 * _R:3 * _R, :]
    ay2 = anch_ref[3 * _R:4 * _R, :]
    dx = delt_ref[0 * _R:1 * _R, :]
    dy = delt_ref[1 * _R:2 * _R, :]
    dw = delt_ref[2 * _R:3 * _R, :]
    dh = delt_ref[3 * _R:4 * _R, :]
    obj = obj_ref[...]

    widths = ax2 - ax1
    heights = ay2 - ay1
    ctr_x = ax1 + 0.5 * widths
    ctr_y = ay1 + 0.5 * heights
    dw = jnp.minimum(dw, _BBOX_XFORM_CLIP)
    dh = jnp.minimum(dh, _BBOX_XFORM_CLIP)
    pred_ctr_x = dx * widths + ctr_x
    pred_ctr_y = dy * heights + ctr_y
    pred_w = jnp.exp(dw) * widths
    pred_h = jnp.exp(dh) * heights

    x1 = jnp.clip(pred_ctr_x - 0.5 * pred_w, 0.0, _IMG_W)
    y1 = jnp.clip(pred_ctr_y - 0.5 * pred_h, 0.0, _IMG_H)
    x2 = jnp.clip(pred_ctr_x + 0.5 * pred_w, 0.0, _IMG_W)
    y2 = jnp.clip(pred_ctr_y + 0.5 * pred_h, 0.0, _IMG_H)

    row = lax.broadcasted_iota(jnp.int32, (_R, _C), 0)
    col = lax.broadcasted_iota(jnp.int32, (_R, _C), 1)
    elem = row * _C + col
    valid = ((x2 - x1) * (y2 - y1) > 1.0) & (elem < _N)
    score = jnp.where(valid, obj, -jnp.inf)

    data_ref[0 * _R:1 * _R, :] = x1
    data_ref[1 * _R:2 * _R, :] = y1
    data_ref[2 * _R:3 * _R, :] = x2
    data_ref[3 * _R:4 * _R, :] = y2
    data_ref[4 * _R:5 * _R, :] = score

    sclean = jnp.where(score == 0.0, 0.0, score)
    bits = lax.bitcast_convert_type(sclean, jnp.uint32)
    flip = jnp.where(bits >= jnp.uint32(0x80000000),
                     jnp.uint32(0xFFFFFFFF), jnp.uint32(0x80000000))
    key = bits ^ flip

    def srch(b, t):
        cand = t | (jnp.uint32(1) << jnp.uint32(31 - b))
        cnt = jnp.sum((key >= cand).astype(jnp.float32))
        return jnp.where(cnt >= float(_PRE_NMS), cand, t)

    t_key = lax.fori_loop(0, 32, srch, jnp.uint32(0))

    gt = (key > t_key).astype(jnp.float32)
    eq = (key == t_key).astype(jnp.float32)
    n_gt = jnp.sum(gt)
    fill = float(_PRE_NMS) - n_gt

    ucol = lax.broadcasted_iota(jnp.int32, (_C, _C), 1)
    urow = lax.broadcasted_iota(jnp.int32, (_C, _C), 0)
    ustrict = (urow < ucol).astype(jnp.float32)
    lcol = lax.broadcasted_iota(jnp.int32, (_R, _R), 1)
    lrow = lax.broadcasted_iota(jnp.int32, (_R, _R), 0)
    lstrict = (lcol < lrow).astype(jnp.float32)
    ones_c = jnp.ones((_C, 1), jnp.float32)

    def excl_prefix(v):
        inrow = lax.dot_general(v, ustrict, (((1,), (0,)), ((), ())),
                                preferred_element_type=jnp.float32)
        rowsum = lax.dot_general(v, ones_c, (((1,), (0,)), ((), ())),
                                 preferred_element_type=jnp.float32)
        rowpref = lax.dot_general(lstrict, rowsum, (((1,), (0,)), ((), ())),
                                  preferred_element_type=jnp.float32)
        return inrow + rowpref

    tie_rank = excl_prefix(eq)
    sel = gt + eq * (tie_rank < fill).astype(jnp.float32)
    dest = excl_prefix(sel)
    dump = (_CPAD + elem).astype(jnp.float32)
    idx = jnp.where(sel > 0.0, dest, dump).astype(jnp.int32)
    idx_ref[...] = idx


def _nms_kernel(cand_ref, candt_ref, out_ref, a_ref):
    x1r = cand_ref[0:1, :]
    y1r = cand_ref[1:2, :]
    x2r = cand_ref[2:3, :]
    y2r = cand_ref[3:4, :]
    scr = cand_ref[4:5, :]
    x1c = candt_ref[:, 0:1]
    y1c = candt_ref[:, 1:2]
    x2c = candt_ref[:, 2:3]
    y2c = candt_ref[:, 3:4]
    sc_c = candt_ref[:, 4:5]

    area_r = (x2r - x1r) * (y2r - y1r)
    area_c = (x2c - x1c) * (y2c - y1c)

    col = lax.broadcasted_iota(jnp.int32, (1, _CPAD), 1)
    row = lax.broadcasted_iota(jnp.int32, (_CPAD, 1), 0)

    blk = 256
    for b in range(_CPAD // blk):
        r0 = b * blk
        bx1 = x1c[r0:r0 + blk, :]
        by1 = y1c[r0:r0 + blk, :]
        bx2 = x2c[r0:r0 + blk, :]
        by2 = y2c[r0:r0 + blk, :]
        bsc = sc_c[r0:r0 + blk, :]
        barea = area_c[r0:r0 + blk, :]
        ltx = jnp.maximum(bx1, x1r)
        lty = jnp.maximum(by1, y1r)
        rbx = jnp.minimum(bx2, x2r)
        rby = jnp.minimum(by2, y2r)
        wx = jnp.clip(rbx - ltx, 0.0, None)
        wy = jnp.clip(rby - lty, 0.0, None)
        inter = wx * wy
        union = barea + area_r - inter
        iou = inter / jnp.maximum(union, 1e-9)
        brow = row[r0:r0 + blk, :]
        outranks = (scr > bsc) | ((scr == bsc) & (col < brow))
        mask = ((iou > _NMS_THRESH) & outranks
                & (col < _PRE_NMS) & (brow < _PRE_NMS))
        a_ref[r0:r0 + blk, :] = mask.astype(jnp.float32)

    valid_row = (row < _PRE_NMS).astype(jnp.float32)

    def cond(carry):
        _, changed, it = carry
        return changed & (it < _CPAD + 2)

    def body(carry):
        k, _, it = carry
        s = lax.dot_general(a_ref[...], k, (((1,), (0,)), ((), ())),
                            preferred_element_type=jnp.float32)
        k_new = jnp.where(s > 0.0, 0.0, valid_row)
        changed = jnp.any(k_new != k)
        return k_new, changed, it + 1

    keep, _, _ = lax.while_loop(cond, body, (valid_row, jnp.bool_(True), jnp.int32(0)))

    m = (keep > 0.0) & (sc_c > -jnp.inf) & (row < _PRE_NMS)
    nm = (~m) & (row < _PRE_NMS)
    mf = m.astype(jnp.float32)
    nmf = nm.astype(jnp.float32)

    for b in range(_CPAD // blk):
        r0 = b * blk
        brow = row[r0:r0 + blk, :]
        bsc = sc_c[r0:r0 + blk, :]
        pm = (((scr > bsc) | ((scr == bsc) & (col < brow)))
              & (col < _PRE_NMS) & (brow < _PRE_NMS))
        a_ref[r0:r0 + blk, :] = pm.astype(jnp.float32)

    cnt_m = lax.dot_general(a_ref[...], mf, (((1,), (0,)), ((), ())),
                            preferred_element_type=jnp.float32)

    for b in range(_CPAD // blk):
        r0 = b * blk
        brow = row[r0:r0 + blk, :]
        a_ref[r0:r0 + blk, :] = ((col < brow) & (col < _PRE_NMS)).astype(jnp.float32)

    cnt_nm = lax.dot_general(a_ref[...], nmf, (((1,), (0,)), ((), ())),
                             preferred_element_type=jnp.float32)

    n_m = jnp.sum(mf)
    rank = jnp.where(m, cnt_m, n_m + cnt_nm)
    rank = jnp.where(row < _PRE_NMS, rank, 2.0 * _CPAD)

    out_col = lax.broadcasted_iota(jnp.int32, (1, _OPAD), 1).astype(jnp.float32)
    sel = (rank == out_col) & (out_col < _POST_NMS)

    neg = -jnp.inf
    score_val = jnp.where(m, sc_c, neg)
    out_ref[0:1, :] = jnp.max(jnp.where(sel, x1c, neg), axis=0, keepdims=True)
    out_ref[1:2, :] = jnp.max(jnp.where(sel, y1c, neg), axis=0, keepdims=True)
    out_ref[2:3, :] = jnp.max(jnp.where(sel, x2c, neg), axis=0, keepdims=True)
    out_ref[3:4, :] = jnp.max(jnp.where(sel, y2c, neg), axis=0, keepdims=True)
    out_ref[4:5, :] = jnp.max(jnp.where(sel, score_val, neg), axis=0, keepdims=True)
    out_ref[5:8, :] = jnp.zeros_like(out_ref[5:8, :])


@functools.lru_cache(maxsize=1)
def _make_sc_scatter():
    mesh = plsc.VectorSubcoreMesh(core_axis_name="c", subcore_axis_name="s")
    col_ty = jax.ShapeDtypeStruct((_SPAD,), jnp.float32)

    @functools.partial(
        pl.kernel, mesh=mesh,
        out_type=tuple(col_ty for _ in range(5)),
        scratch_types=[
            pltpu.VMEM((_CHUNK,), jnp.int32),
            pltpu.VMEM((_CHUNK,), jnp.float32),
            pltpu.VMEM((_CHUNK,), jnp.float32),
            pltpu.VMEM((_CHUNK,), jnp.float32),
            pltpu.VMEM((_CHUNK,), jnp.float32),
            pltpu.VMEM((_CHUNK,), jnp.float32),
            pltpu.SemaphoreType.DMA,
        ],
    )
    def sc_scatter(idx_hbm, x1_hbm, y1_hbm, x2_hbm, y2_hbm, sc_hbm,
                   o0, o1, o2, o3, o4, idx_v, v0, v1, v2, v3, v4, sem):
        wid = lax.axis_index("s") * 2 + lax.axis_index("c")
        base = wid * _CHUNK
        pltpu.sync_copy(idx_hbm.at[pl.ds(base, _CHUNK)], idx_v)
        pltpu.sync_copy(x1_hbm.at[pl.ds(base, _CHUNK)], v0)
        pltpu.sync_copy(y1_hbm.at[pl.ds(base, _CHUNK)], v1)
        pltpu.sync_copy(x2_hbm.at[pl.ds(base, _CHUNK)], v2)
        pltpu.sync_copy(y2_hbm.at[pl.ds(base, _CHUNK)], v3)
        pltpu.sync_copy(sc_hbm.at[pl.ds(base, _CHUNK)], v4)
        c0 = pltpu.async_copy(v0, o0.at[idx_v], sem)
        c1 = pltpu.async_copy(v1, o1.at[idx_v], sem)
        c2 = pltpu.async_copy(v2, o2.at[idx_v], sem)
        c3 = pltpu.async_copy(v3, o3.at[idx_v], sem)
        c4 = pltpu.async_copy(v4, o4.at[idx_v], sem)
        c0.wait()
        c1.wait()
        c2.wait()
        c3.wait()
        c4.wait()

    return sc_scatter


def _decode_select(anchors, pred_bbox_deltas, objectness, interpret=False):
    f32 = jnp.float32
    n_extra = _NPAD - _N
    at = anchors.T.astype(f32)
    dt = pred_bbox_deltas.T.astype(f32)
    at = jnp.pad(at, ((0, 0), (0, n_extra))).reshape(4 * _R, _C)
    dt = jnp.pad(dt, ((0, 0), (0, n_extra))).reshape(4 * _R, _C)
    ob = jnp.pad(objectness.astype(f32), (0, n_extra)).reshape(_R, _C)

    data, idx = pl.pallas_call(
        _decode_select_kernel,
        out_shape=(jax.ShapeDtypeStruct((5 * _R, _C), f32),
                   jax.ShapeDtypeStruct((_R, _C), jnp.int32)),
        interpret=interpret,
    )(at, dt, ob)
    return data, idx


def _nms(cols, interpret=False):
    f32 = jnp.float32
    cand = jnp.stack([c[:_CPAD] for c in cols])
    cand = jnp.pad(cand, ((0, 3), (0, 0)))
    candt = cand.T
    out = pl.pallas_call(
        _nms_kernel,
        out_shape=jax.ShapeDtypeStruct((8, _OPAD), f32),
        scratch_shapes=[pltpu.VMEM((_CPAD, _CPAD), f32)],
        interpret=interpret,
    )(cand, candt)
    return out[0:5, :_POST_NMS].T


@jax.jit
def _run(anchors, pred_bbox_deltas, objectness):
    data, idx = _decode_select(anchors, pred_bbox_deltas, objectness)
    planes = [data[i * _R:(i + 1) * _R, :].reshape(_NPAD) for i in range(5)]
    cols = _make_sc_scatter()(idx.reshape(_NPAD), *planes)
    return _nms(cols)


def kernel(anchors, pred_bbox_deltas, objectness):
    return _run(anchors, pred_bbox_deltas, objectness)
